# baseline (speedup 1.0000x reference)
import functools

import jax
import jax.numpy as jnp
from jax.experimental import pallas as pl
from jax.experimental.pallas import tpu as pltpu

N = 10000
E = 160000
D = 512
H = 8
C = 64
HC = H * C
ED = 7
G = 64
NEG_SLOPE = 0.2


def _mm2_body(x_ref, wl_ref, wr_ref, bl_ref, br_ref, xl_ref, xr_ref):
    x = x_ref[...]
    xl_ref[...] = (
        jnp.dot(x, wl_ref[...], preferred_element_type=jnp.float32) + bl_ref[...]
    )
    xr_ref[...] = (
        jnp.dot(x, wr_ref[...], preferred_element_type=jnp.float32) + br_ref[...]
    )


def _dual_project(x, Wl, bl, Wr, br):
    nrows = x.shape[0]
    blk = 400
    grid = (nrows // blk,)
    return pl.pallas_call(
        _mm2_body,
        grid=grid,
        in_specs=[
            pl.BlockSpec((blk, D), lambda i: (i, 0)),
            pl.BlockSpec((D, HC), lambda i: (0, 0)),
            pl.BlockSpec((D, HC), lambda i: (0, 0)),
            pl.BlockSpec((1, HC), lambda i: (0, 0)),
            pl.BlockSpec((1, HC), lambda i: (0, 0)),
        ],
        out_specs=[
            pl.BlockSpec((blk, HC), lambda i: (i, 0)),
            pl.BlockSpec((blk, HC), lambda i: (i, 0)),
        ],
        out_shape=[
            jax.ShapeDtypeStruct((nrows, HC), jnp.float32),
            jax.ShapeDtypeStruct((nrows, HC), jnp.float32),
        ],
    )(x, Wl, Wr, bl.reshape(1, HC), br.reshape(1, HC))


def _edge_mm_body(a_ref, w_ref, b_ref, o_ref):
    o_ref[...] = (
        jnp.dot(a_ref[...], w_ref[...], preferred_element_type=jnp.float32)
        + b_ref[...]
    )


def _edge_project(edge_attr, We, be):
    a = jnp.pad(edge_attr, ((0, 0), (0, 8 - ED)))
    w = jnp.pad(We, ((0, 8 - ED), (0, 0)))
    blk = 2000
    return pl.pallas_call(
        _edge_mm_body,
        grid=(E // blk,),
        in_specs=[
            pl.BlockSpec((blk, 8), lambda i: (i, 0)),
            pl.BlockSpec((8, HC), lambda i: (0, 0)),
            pl.BlockSpec((1, HC), lambda i: (0, 0)),
        ],
        out_specs=pl.BlockSpec((blk, HC), lambda i: (i, 0)),
        out_shape=jax.ShapeDtypeStruct((E, HC), jnp.float32),
    )(a, w, be.reshape(1, HC))


def _pool_body(h_ref, oh_ref, sums_ref, cnt_ref):
    i = pl.program_id(0)

    @pl.when(i == 0)
    def _init():
        sums_ref[...] = jnp.zeros_like(sums_ref)
        cnt_ref[...] = jnp.zeros_like(cnt_ref)

    oh = oh_ref[...]
    sums_ref[...] += jnp.dot(
        oh.T, h_ref[...], preferred_element_type=jnp.float32
    )
    cnt_ref[...] += jnp.sum(oh, axis=0, keepdims=True)


def _mean_pool(h, batch, bias):
    blk = 400
    onehot = (batch[:, None] == jnp.arange(G)[None, :]).astype(jnp.float32)
    sums, cnt = pl.pallas_call(
        _pool_body,
        grid=(N // blk,),
        in_specs=[
            pl.BlockSpec((blk, HC), lambda i: (i, 0)),
            pl.BlockSpec((blk, G), lambda i: (i, 0)),
        ],
        out_specs=[
            pl.BlockSpec((G, HC), lambda i: (0, 0)),
            pl.BlockSpec((1, G), lambda i: (0, 0)),
        ],
        out_shape=[
            jax.ShapeDtypeStruct((G, HC), jnp.float32),
            jax.ShapeDtypeStruct((1, G), jnp.float32),
        ],
    )(h, onehot)
    return sums / jnp.maximum(cnt[0], 1.0)[:, None] + bias[None, :]


def _sparse_attention(xl, xr, e, src, dst):
    m = xl.reshape(N, H, C)[src] + xr.reshape(N, H, C)[dst] + e.reshape(E, H, C)
    m = jnp.where(m > 0, m, NEG_SLOPE * m)
    return m


def _gat_layer(x, src, dst, e, Wl, bl, Wr, br, att):
    xl, xr = _dual_project(x, Wl, bl, Wr, br)
    m = _sparse_attention(xl, xr, e, src, dst)
    alpha = jnp.einsum("ehc,hc->eh", m, att)
    amax = jax.ops.segment_max(alpha, dst, num_segments=N)
    amax = jnp.where(jnp.isfinite(amax), amax, 0.0)
    ex = jnp.exp(alpha - amax[dst])
    denom = jax.ops.segment_sum(ex, dst, num_segments=N)
    a = ex / (denom[dst] + 1e-16)
    out = jax.ops.segment_sum(
        xl.reshape(N, H, C)[src] * a[:, :, None], dst, num_segments=N
    )
    return out.reshape(N, HC)


def kernel(x, edge_index, edge_attr, batch, Wl1, bl1, Wr1, br1, We1, be1,
           att1, bias1, Wl2, bl2, Wr2, br2, We2, be2, att2, bias2):
    src = edge_index[0]
    dst = edge_index[1]
    e1 = _edge_project(edge_attr, We1, be1)
    e2 = _edge_project(edge_attr, We2, be2)
    h = _gat_layer(x, src, dst, e1, Wl1, bl1, Wr1, br1, att1) + bias1
    h2 = _gat_layer(h, src, dst, e2, Wl2, bl2, Wr2, br2, att2)
    return _mean_pool(h2, batch, bias2)


# speedup vs baseline: 1.0947x; 1.0947x over previous
import dataclasses
import functools

import jax
import jax.numpy as jnp
from jax import lax
from jax.experimental import pallas as pl
from jax.experimental.pallas import tpu as pltpu
from jax.experimental.pallas import tpu_sc as plsc

N = 10000
E = 160000
D = 512
H = 8
C = 64
HC = H * C
ED = 7
G = 64
NEG_SLOPE = 0.2


def _mm2_body(x_ref, wl_ref, wr_ref, bl_ref, br_ref, xl_ref, xr_ref):
    x = x_ref[...]
    xl_ref[...] = (
        jnp.dot(x, wl_ref[...], preferred_element_type=jnp.float32) + bl_ref[...]
    )
    xr_ref[...] = (
        jnp.dot(x, wr_ref[...], preferred_element_type=jnp.float32) + br_ref[...]
    )


def _dual_project(x, Wl, bl, Wr, br):
    nrows = x.shape[0]
    blk = 400
    grid = (nrows // blk,)
    return pl.pallas_call(
        _mm2_body,
        grid=grid,
        in_specs=[
            pl.BlockSpec((blk, D), lambda i: (i, 0)),
            pl.BlockSpec((D, HC), lambda i: (0, 0)),
            pl.BlockSpec((D, HC), lambda i: (0, 0)),
            pl.BlockSpec((1, HC), lambda i: (0, 0)),
            pl.BlockSpec((1, HC), lambda i: (0, 0)),
        ],
        out_specs=[
            pl.BlockSpec((blk, HC), lambda i: (i, 0)),
            pl.BlockSpec((blk, HC), lambda i: (i, 0)),
        ],
        out_shape=[
            jax.ShapeDtypeStruct((nrows, HC), jnp.float32),
            jax.ShapeDtypeStruct((nrows, HC), jnp.float32),
        ],
    )(x, Wl, Wr, bl.reshape(1, HC), br.reshape(1, HC))


def _edge_mm_body(a_ref, w_ref, b_ref, o_ref):
    o_ref[...] = (
        jnp.dot(a_ref[...], w_ref[...], preferred_element_type=jnp.float32)
        + b_ref[...]
    )


def _edge_project(edge_attr, We, be):
    a = jnp.pad(edge_attr, ((0, 0), (0, 8 - ED)))
    w = jnp.pad(We, ((0, 8 - ED), (0, 0)))
    blk = 2000
    return pl.pallas_call(
        _edge_mm_body,
        grid=(E // blk,),
        in_specs=[
            pl.BlockSpec((blk, 8), lambda i: (i, 0)),
            pl.BlockSpec((8, HC), lambda i: (0, 0)),
            pl.BlockSpec((1, HC), lambda i: (0, 0)),
        ],
        out_specs=pl.BlockSpec((blk, HC), lambda i: (i, 0)),
        out_shape=jax.ShapeDtypeStruct((E, HC), jnp.float32),
    )(a, w, be.reshape(1, HC))


def _pool_body(h_ref, oh_ref, sums_ref, cnt_ref):
    i = pl.program_id(0)

    @pl.when(i == 0)
    def _init():
        sums_ref[...] = jnp.zeros_like(sums_ref)
        cnt_ref[...] = jnp.zeros_like(cnt_ref)

    oh = oh_ref[...]
    sums_ref[...] += jnp.dot(
        oh.T, h_ref[...], preferred_element_type=jnp.float32
    )
    cnt_ref[...] += jnp.sum(oh, axis=0, keepdims=True)


def _mean_pool(h, batch, bias):
    blk = 400
    onehot = (batch[:, None] == jnp.arange(G)[None, :]).astype(jnp.float32)
    sums, cnt = pl.pallas_call(
        _pool_body,
        grid=(N // blk,),
        in_specs=[
            pl.BlockSpec((blk, HC), lambda i: (i, 0)),
            pl.BlockSpec((blk, G), lambda i: (i, 0)),
        ],
        out_specs=[
            pl.BlockSpec((G, HC), lambda i: (0, 0)),
            pl.BlockSpec((1, G), lambda i: (0, 0)),
        ],
        out_shape=[
            jax.ShapeDtypeStruct((G, HC), jnp.float32),
            jax.ShapeDtypeStruct((1, G), jnp.float32),
        ],
    )(h, onehot)
    return sums / jnp.maximum(cnt[0], 1.0)[:, None] + bias[None, :]


NWORK = 32
EPW = E // NWORK
ABLK = 40
NEG = -1e30

_SC_MESH = dict(core_axis_name="c", subcore_axis_name="s")

_SC_PARAMS = pltpu.CompilerParams()
if "needs_layout_passes" in pltpu.CompilerParams.__dataclass_fields__:
    _SC_PARAMS = dataclasses.replace(_SC_PARAMS, needs_layout_passes=False)


def _alpha_body(xl_hbm, xr_hbm, e_hbm, src_hbm, dst_hbm, att_hbm,
                alpha_hbm, tmax_hbm,
                sidx_v, didx_v, xl_v, xr_v, e_v, alpha_v, att_v, maxv_v, sem):
    wid = lax.axis_index("s") * 2 + lax.axis_index("c")
    base_t = wid * EPW
    pltpu.sync_copy(att_hbm, att_v)
    maxv_v[...] = jnp.full((16,), NEG, jnp.float32)

    @pl.loop(0, EPW // ABLK)
    def _blk(b):
        base = base_t + b * ABLK
        pltpu.sync_copy(src_hbm.at[pl.ds(base, ABLK)], sidx_v)
        pltpu.sync_copy(dst_hbm.at[pl.ds(base, ABLK)], didx_v)
        c1 = pltpu.async_copy(xl_hbm.at[sidx_v], xl_v, sem)
        c2 = pltpu.async_copy(xr_hbm.at[didx_v], xr_v, sem)
        c3 = pltpu.async_copy(e_hbm.at[pl.ds(base, ABLK)], e_v, sem)
        c1.wait()
        c2.wait()
        c3.wait()

        lane = lax.iota(jnp.int32, 16)

        @pl.loop(0, ABLK)
        def _edge(j):
            row = jnp.full((16,), NEG, jnp.float32)
            for h in range(H):
                acc = jnp.zeros((16,), jnp.float32)
                for q in range(C // 16):
                    sl = pl.ds(h * C + q * 16, 16)
                    m = xl_v[j, sl] + xr_v[j, sl] + e_v[j, sl]
                    m = jnp.where(m > 0, m, NEG_SLOPE * m)
                    acc = acc + m * att_v[h, pl.ds(q * 16, 16)]
                row = jnp.where(lane == h, jnp.sum(acc), row)
            alpha_v[j, pl.ds(0, 16)] = row

        @pl.loop(0, ABLK)
        def _mx(j):
            maxv_v[...] = jnp.maximum(maxv_v[...], alpha_v[j, pl.ds(0, 16)])

        pltpu.sync_copy(alpha_v, alpha_hbm.at[pl.ds(base, ABLK)])

    pltpu.sync_copy(maxv_v, tmax_hbm.at[wid])


def _sc_alpha(xl, xr, e, src, dst, att):
    kern = functools.partial(
        pl.kernel,
        mesh=plsc.VectorSubcoreMesh(**_SC_MESH),
        compiler_params=_SC_PARAMS,
        out_type=[
            jax.ShapeDtypeStruct((E, 16), jnp.float32),
            jax.ShapeDtypeStruct((NWORK, 16), jnp.float32),
        ],
        scratch_types=[
            pltpu.VMEM((ABLK,), jnp.int32),
            pltpu.VMEM((ABLK,), jnp.int32),
            pltpu.VMEM((ABLK, HC), jnp.float32),
            pltpu.VMEM((ABLK, HC), jnp.float32),
            pltpu.VMEM((ABLK, HC), jnp.float32),
            pltpu.VMEM((ABLK, 16), jnp.float32),
            pltpu.VMEM((H, C), jnp.float32),
            pltpu.VMEM((16,), jnp.float32),
            pltpu.SemaphoreType.DMA,
        ],
    )(_alpha_body)
    return kern(xl, xr, e, src, dst, att)


def _gmax_body(t_ref, o_ref):
    o_ref[...] = jnp.max(t_ref[...]).reshape(1, 1)


def _gmax(tmax):
    return pl.pallas_call(
        _gmax_body,
        out_shape=jax.ShapeDtypeStruct((1, 1), jnp.float32),
    )(tmax)[0, 0]


def _gat_layer(x, src, dst, e, Wl, bl, Wr, br, att):
    xl, xr = _dual_project(x, Wl, bl, Wr, br)
    alpha16, tmax = _sc_alpha(xl, xr, e, src, dst, att)
    gmax = _gmax(tmax)
    alpha = alpha16[:, :H]
    ex = jnp.exp(alpha - gmax)
    denom = jax.ops.segment_sum(ex, dst, num_segments=N)
    a = ex / (denom[dst] + 1e-16)
    out = jax.ops.segment_sum(
        xl.reshape(N, H, C)[src] * a[:, :, None], dst, num_segments=N
    )
    return out.reshape(N, HC)


def kernel(x, edge_index, edge_attr, batch, Wl1, bl1, Wr1, br1, We1, be1,
           att1, bias1, Wl2, bl2, Wr2, br2, We2, be2, att2, bias2):
    src = edge_index[0]
    dst = edge_index[1]
    e1 = _edge_project(edge_attr, We1, be1)
    e2 = _edge_project(edge_attr, We2, be2)
    h = _gat_layer(x, src, dst, e1, Wl1, bl1, Wr1, br1, att1) + bias1
    h2 = _gat_layer(h, src, dst, e2, Wl2, bl2, Wr2, br2, att2)
    return _mean_pool(h2, batch, bias2)


# speedup vs baseline: 7.3633x; 6.7266x over previous
import dataclasses
import functools

import jax
import jax.numpy as jnp
from jax import lax
from jax.experimental import pallas as pl
from jax.experimental.pallas import tpu as pltpu
from jax.experimental.pallas import tpu_sc as plsc

N = 10000
E = 160000
D = 512
H = 8
C = 64
HC = H * C
ED = 7
G = 64
NEG_SLOPE = 0.2


def _mm2_body(x_ref, wl_ref, wr_ref, bl_ref, br_ref, xl_ref, xr_ref):
    x = x_ref[...]
    xl_ref[...] = (
        jnp.dot(x, wl_ref[...], preferred_element_type=jnp.float32) + bl_ref[...]
    )
    xr_ref[...] = (
        jnp.dot(x, wr_ref[...], preferred_element_type=jnp.float32) + br_ref[...]
    )


def _dual_project(x, Wl, bl, Wr, br):
    nrows = x.shape[0]
    blk = 400
    grid = (nrows // blk,)
    return pl.pallas_call(
        _mm2_body,
        grid=grid,
        in_specs=[
            pl.BlockSpec((blk, D), lambda i: (i, 0)),
            pl.BlockSpec((D, HC), lambda i: (0, 0)),
            pl.BlockSpec((D, HC), lambda i: (0, 0)),
            pl.BlockSpec((1, HC), lambda i: (0, 0)),
            pl.BlockSpec((1, HC), lambda i: (0, 0)),
        ],
        out_specs=[
            pl.BlockSpec((blk, HC), lambda i: (i, 0)),
            pl.BlockSpec((blk, HC), lambda i: (i, 0)),
        ],
        out_shape=[
            jax.ShapeDtypeStruct((nrows, HC), jnp.float32),
            jax.ShapeDtypeStruct((nrows, HC), jnp.float32),
        ],
    )(x, Wl, Wr, bl.reshape(1, HC), br.reshape(1, HC))


def _edge_mm_body(a_ref, w_ref, b_ref, o_ref):
    o_ref[...] = (
        jnp.dot(a_ref[...], w_ref[...], preferred_element_type=jnp.float32)
        + b_ref[...]
    )


def _edge_project(edge_attr, We, be):
    a = jnp.pad(edge_attr, ((0, 0), (0, 8 - ED)))
    w = jnp.pad(We, ((0, 8 - ED), (0, 0)))
    blk = 2000
    return pl.pallas_call(
        _edge_mm_body,
        grid=(E // blk,),
        in_specs=[
            pl.BlockSpec((blk, 8), lambda i: (i, 0)),
            pl.BlockSpec((8, HC), lambda i: (0, 0)),
            pl.BlockSpec((1, HC), lambda i: (0, 0)),
        ],
        out_specs=pl.BlockSpec((blk, HC), lambda i: (i, 0)),
        out_shape=jax.ShapeDtypeStruct((E, HC), jnp.float32),
    )(a, w, be.reshape(1, HC))


def _pool_body(h_ref, oh_ref, sums_ref, cnt_ref):
    i = pl.program_id(0)

    @pl.when(i == 0)
    def _init():
        sums_ref[...] = jnp.zeros_like(sums_ref)
        cnt_ref[...] = jnp.zeros_like(cnt_ref)

    oh = oh_ref[...]
    sums_ref[...] += jnp.dot(
        oh.T, h_ref[...], preferred_element_type=jnp.float32
    )
    cnt_ref[...] += jnp.sum(oh, axis=0, keepdims=True)


def _mean_pool(h, batch, bias):
    blk = 400
    onehot = (batch[:, None] == jnp.arange(G)[None, :]).astype(jnp.float32)
    sums, cnt = pl.pallas_call(
        _pool_body,
        grid=(N // blk,),
        in_specs=[
            pl.BlockSpec((blk, HC), lambda i: (i, 0)),
            pl.BlockSpec((blk, G), lambda i: (i, 0)),
        ],
        out_specs=[
            pl.BlockSpec((G, HC), lambda i: (0, 0)),
            pl.BlockSpec((1, G), lambda i: (0, 0)),
        ],
        out_shape=[
            jax.ShapeDtypeStruct((G, HC), jnp.float32),
            jax.ShapeDtypeStruct((1, G), jnp.float32),
        ],
    )(h, onehot)
    return sums / jnp.maximum(cnt[0], 1.0)[:, None] + bias[None, :]


NWORK = 32
EPW = E // NWORK
ABLK = 40
NEG = -1e30

_SC_MESH = dict(core_axis_name="c", subcore_axis_name="s")

_SC_PARAMS = pltpu.CompilerParams()
if "needs_layout_passes" in pltpu.CompilerParams.__dataclass_fields__:
    _SC_PARAMS = dataclasses.replace(_SC_PARAMS, needs_layout_passes=False)


def _alpha_body(xl_hbm, xr_hbm, e_hbm, src_hbm, dst_hbm, att_hbm,
                alpha_hbm, tmax_hbm,
                sidx_v, didx_v, xl_v, xr_v, e_v, alpha_v, att_v, maxv_v, sem):
    wid = lax.axis_index("s") * 2 + lax.axis_index("c")
    base_t = wid * EPW
    pltpu.sync_copy(att_hbm, att_v)
    maxv_v[...] = jnp.full((16,), NEG, jnp.float32)

    @pl.loop(0, EPW // ABLK)
    def _blk(b):
        base = base_t + b * ABLK
        pltpu.sync_copy(src_hbm.at[pl.ds(base, ABLK)], sidx_v)
        pltpu.sync_copy(dst_hbm.at[pl.ds(base, ABLK)], didx_v)
        c1 = pltpu.async_copy(xl_hbm.at[sidx_v], xl_v, sem)
        c2 = pltpu.async_copy(xr_hbm.at[didx_v], xr_v, sem)
        c3 = pltpu.async_copy(e_hbm.at[pl.ds(base, ABLK)], e_v, sem)
        c1.wait()
        c2.wait()
        c3.wait()

        lane = lax.iota(jnp.int32, 16)

        @pl.loop(0, ABLK)
        def _edge(j):
            row = jnp.full((16,), NEG, jnp.float32)
            for h in range(H):
                acc = jnp.zeros((16,), jnp.float32)
                for q in range(C // 16):
                    sl = pl.ds(h * C + q * 16, 16)
                    m = xl_v[j, sl] + xr_v[j, sl] + e_v[j, sl]
                    m = jnp.where(m > 0, m, NEG_SLOPE * m)
                    acc = acc + m * att_v[h, pl.ds(q * 16, 16)]
                row = jnp.where(lane == h, jnp.sum(acc), row)
            alpha_v[j, pl.ds(0, 16)] = row

        @pl.loop(0, ABLK)
        def _mx(j):
            maxv_v[...] = jnp.maximum(maxv_v[...], alpha_v[j, pl.ds(0, 16)])

        pltpu.sync_copy(alpha_v, alpha_hbm.at[pl.ds(base, ABLK)])

    pltpu.sync_copy(maxv_v, tmax_hbm.at[wid])


def _sc_alpha(xl, xr, e, src, dst, att):
    kern = functools.partial(
        pl.kernel,
        mesh=plsc.VectorSubcoreMesh(**_SC_MESH),
        compiler_params=_SC_PARAMS,
        out_type=[
            jax.ShapeDtypeStruct((E, 128), jnp.float32),
            jax.ShapeDtypeStruct((NWORK, 16), jnp.float32),
        ],
        scratch_types=[
            pltpu.VMEM((ABLK,), jnp.int32),
            pltpu.VMEM((ABLK,), jnp.int32),
            pltpu.VMEM((ABLK, HC), jnp.float32),
            pltpu.VMEM((ABLK, HC), jnp.float32),
            pltpu.VMEM((ABLK, HC), jnp.float32),
            pltpu.VMEM((ABLK, 128), jnp.float32),
            pltpu.VMEM((H, C), jnp.float32),
            pltpu.VMEM((16,), jnp.float32),
            pltpu.SemaphoreType.DMA,
        ],
    )(_alpha_body)
    return kern(xl, xr, e, src, dst, att)


NB = 256
BN = 40
TBL = E + 16
SLC1 = 5008
SLC0 = E - (NWORK - 1) * SLC1
BL0 = SLC0 // 16
BL1 = SLC1 // 16


def _hist_body(dst_hbm, hist_hbm, dstb_v, hist_v):
    wid = lax.axis_index("s") * 2 + lax.axis_index("c")
    base = wid * SLC1
    pltpu.sync_copy(dst_hbm.at[pl.ds(base, SLC0)], dstb_v.at[pl.ds(0, SLC0)])

    @pl.when(wid < NWORK - 1)
    def _ld_tail():
        pltpu.sync_copy(dst_hbm.at[pl.ds(base + SLC0, SLC1 - SLC0)],
                        dstb_v.at[pl.ds(SLC0, SLC1 - SLC0)])

    lane = lax.iota(jnp.int32, 16)
    zi = jnp.zeros((16,), jnp.int32)
    for i in range(NB // 16):
        hist_v[pl.ds(i * 16, 16)] = zi

    def _blk(b):
        bv = dstb_v[pl.ds(b * 16, 16)] // BN
        for l in range(16):
            b_l = bv[l]
            fb = (b_l // 16) * 16
            w = hist_v[pl.ds(fb, 16)]
            hist_v[pl.ds(fb, 16)] = w + jnp.where(lane == b_l - fb, 1, 0)

    pl.loop(0, BL0)(_blk)

    @pl.when(wid < NWORK - 1)
    def _tail():
        pl.loop(BL0, BL1)(_blk)

    pltpu.sync_copy(hist_v, hist_hbm.at[wid])


def _sc_hist(dst):
    kern = functools.partial(
        pl.kernel,
        mesh=plsc.VectorSubcoreMesh(**_SC_MESH),
        compiler_params=_SC_PARAMS,
        out_type=jax.ShapeDtypeStruct((NWORK, NB), jnp.int32),
        scratch_types=[
            pltpu.VMEM((SLC1,), jnp.int32),
            pltpu.VMEM((NB,), jnp.int32),
        ],
    )(_hist_body)
    return kern(dst)


def _sort_body(src_hbm, dst_hbm, hist_hbm, table_hbm, bst_hbm,
               srcb_v, dstb_v, h32_v, offs_v, trip_v, pos_v, bst_v, sem):
    wid = lax.axis_index("s") * 2 + lax.axis_index("c")
    base = wid * SLC1
    pltpu.sync_copy(src_hbm.at[pl.ds(base, SLC0)], srcb_v.at[pl.ds(0, SLC0)])
    pltpu.sync_copy(dst_hbm.at[pl.ds(base, SLC0)], dstb_v.at[pl.ds(0, SLC0)])

    @pl.when(wid < NWORK - 1)
    def _ld_tail():
        pltpu.sync_copy(src_hbm.at[pl.ds(base + SLC0, SLC1 - SLC0)],
                        srcb_v.at[pl.ds(SLC0, SLC1 - SLC0)])
        pltpu.sync_copy(dst_hbm.at[pl.ds(base + SLC0, SLC1 - SLC0)],
                        dstb_v.at[pl.ds(SLC0, SLC1 - SLC0)])

    pltpu.sync_copy(hist_hbm, h32_v)
    lane = lax.iota(jnp.int32, 16)

    run = jnp.int32(0)
    for i in range(NB // 16):
        tot = jnp.zeros((16,), jnp.int32)
        pre = jnp.zeros((16,), jnp.int32)
        for w in range(NWORK):
            row = h32_v[w, pl.ds(i * 16, 16)]
            tot = tot + row
            pre = pre + jnp.where(wid > w, row, 0)
        cs = plsc.cumsum(tot)
        excl = cs - tot
        offs_v[pl.ds(i * 16, 16)] = run + excl + pre
        startv = run + excl
        endv = run + cs
        for l in range(16):
            bst_v[i * 16 + l, pl.ds(0, 16)] = (
                jnp.where(lane == 0, startv[l], 0)
                + jnp.where(lane == 1, endv[l], 0)
            )
        run = run + cs[15]

    @pl.when(wid == 0)
    def _aux():
        pltpu.sync_copy(bst_v, bst_hbm)

        @pl.loop(0, 16)
        def _sr(j):
            trip_v[j, pl.ds(0, 16)] = jnp.where(lane == 1, -1, 0)

        pltpu.sync_copy(trip_v, table_hbm.at[pl.ds(E, 16)])

    def _blk(b):
        dv = dstb_v[pl.ds(b * 16, 16)]
        sv = srcb_v[pl.ds(b * 16, 16)]
        bv = dv // BN
        eidv = base + b * 16 + lane
        posacc = jnp.zeros((16,), jnp.int32)
        for l in range(16):
            b_l = bv[l]
            fb = (b_l // 16) * 16
            w = offs_v[pl.ds(fb, 16)]
            sel = lane == (b_l - fb)
            pos_l = jnp.max(jnp.where(sel, w, -1))
            offs_v[pl.ds(fb, 16)] = w + jnp.where(sel, 1, 0)
            posacc = jnp.where(lane == l, pos_l, posacc)
            trip_v[l, pl.ds(0, 16)] = (
                jnp.where(lane == 0, sv[l], 0)
                + jnp.where(lane == 1, dv[l], 0)
                + jnp.where(lane == 2, eidv[l], 0)
            )
        pos_v[0, pl.ds(0, 16)] = posacc
        pltpu.async_copy(trip_v, table_hbm.at[pos_v.at[0]], sem).wait()

    pl.loop(0, BL0)(_blk)

    @pl.when(wid < NWORK - 1)
    def _tail():
        pl.loop(BL0, BL1)(_blk)


def _sc_sort(src, dst, hist):
    kern = functools.partial(
        pl.kernel,
        mesh=plsc.VectorSubcoreMesh(**_SC_MESH),
        compiler_params=_SC_PARAMS,
        out_type=[
            jax.ShapeDtypeStruct((TBL, 128), jnp.int32),
            jax.ShapeDtypeStruct((NB, 16), jnp.int32),
        ],
        scratch_types=[
            pltpu.VMEM((SLC1,), jnp.int32),
            pltpu.VMEM((SLC1,), jnp.int32),
            pltpu.VMEM((NWORK, NB), jnp.int32),
            pltpu.VMEM((NB,), jnp.int32),
            pltpu.VMEM((16, 128), jnp.int32),
            pltpu.VMEM((1, 16), jnp.int32),
            pltpu.VMEM((NB, 16), jnp.int32),
            pltpu.SemaphoreType.DMA,
        ],
    )(_sort_body)
    return kern(src, dst, hist)


def _agg_body(alpha_hbm, xl_hbm, table_hbm, bst_hbm, gv_hbm, out_hbm,
              bst_v, trip_v, eix_v, six_v, axr_v, xlr_v, acc_v, den_v,
              gv_v, sem):
    cid = lax.axis_index("c")
    sid = lax.axis_index("s")
    pltpu.sync_copy(gv_hbm, gv_v)
    pltpu.sync_copy(bst_hbm, bst_v)
    gvreg = gv_v[...]
    lane = lax.iota(jnp.int32, 16)
    zf = jnp.zeros((16,), jnp.float32)

    @pl.loop(0, NB // NWORK)
    def _bk(k):
        b = (cid * (NB // NWORK) + k) * 16 + sid
        brow = bst_v[b, pl.ds(0, 16)]
        start = brow[0]
        end = brow[1]
        lo = b * BN

        @pl.loop(0, BN)
        def _z(r):
            for q in range(HC // 16):
                acc_v[r, pl.ds(q * 16, 16)] = zf
            den_v[r, pl.ds(0, 16)] = zf

        a0 = (start // 16) * 16

        @pl.loop(0, (end - a0 + 15) // 16)
        def _blk(g):
            pltpu.sync_copy(table_hbm.at[pl.ds(a0 + g * 16, 16)], trip_v)
            ei = jnp.zeros((16,), jnp.int32)
            si = jnp.zeros((16,), jnp.int32)
            for l in range(16):
                tr = trip_v[l, pl.ds(0, 16)]
                ei = jnp.where(lane == l, tr[2], ei)
                si = jnp.where(lane == l, tr[0], si)
            eix_v[...] = ei
            six_v[...] = si
            c1 = pltpu.async_copy(alpha_hbm.at[eix_v], axr_v, sem)
            c2 = pltpu.async_copy(xl_hbm.at[six_v], xlr_v, sem)
            c1.wait()
            c2.wait()

            @pl.loop(0, 16)
            def _edge(j):
                dl = trip_v[j, pl.ds(0, 16)][1] - lo

                @pl.when((dl >= 0) & (dl < BN))
                def _proc():
                    ex = jnp.exp(axr_v[j, pl.ds(0, 16)] - gvreg)
                    den_v[dl, pl.ds(0, 16)] = den_v[dl, pl.ds(0, 16)] + ex
                    for h in range(H):
                        a_s = ex[h]
                        for q in range(C // 16):
                            sl = pl.ds(h * C + q * 16, 16)
                            acc_v[dl, sl] = (acc_v[dl, sl]
                                             + xlr_v[j, sl] * a_s)

        @pl.when(lo < N)
        def _dump():
            @pl.loop(0, BN)
            def _n(r):
                invv = 1.0 / (den_v[r, pl.ds(0, 16)] + 1e-16)
                for h in range(H):
                    iv = invv[h]
                    for q in range(C // 16):
                        sl = pl.ds(h * C + q * 16, 16)
                        acc_v[r, sl] = acc_v[r, sl] * iv

            pltpu.sync_copy(acc_v, out_hbm.at[pl.ds(lo, BN)])


def _sc_aggregate(alpha128, xl, table, bst, gvec):
    kern = functools.partial(
        pl.kernel,
        mesh=plsc.VectorSubcoreMesh(**_SC_MESH),
        compiler_params=_SC_PARAMS,
        out_type=jax.ShapeDtypeStruct((N, HC), jnp.float32),
        scratch_types=[
            pltpu.VMEM((NB, 16), jnp.int32),
            pltpu.VMEM((16, 128), jnp.int32),
            pltpu.VMEM((16,), jnp.int32),
            pltpu.VMEM((16,), jnp.int32),
            pltpu.VMEM((16, 128), jnp.float32),
            pltpu.VMEM((16, HC), jnp.float32),
            pltpu.VMEM((BN, HC), jnp.float32),
            pltpu.VMEM((BN, 16), jnp.float32),
            pltpu.VMEM((16,), jnp.float32),
            pltpu.SemaphoreType.DMA,
        ],
    )(_agg_body)
    return kern(alpha128, xl, table, bst, gvec)


def _gmax_body(t_ref, o_ref):
    o_ref[...] = jnp.max(t_ref[...]).reshape(1, 1)


def _gmax(tmax):
    return pl.pallas_call(
        _gmax_body,
        out_shape=jax.ShapeDtypeStruct((1, 1), jnp.float32),
    )(tmax)[0, 0]


def _gat_layer(x, src, dst, e, table, bst, Wl, bl, Wr, br, att):
    xl, xr = _dual_project(x, Wl, bl, Wr, br)
    alpha128, tmax = _sc_alpha(xl, xr, e, src, dst, att)
    gvec = jnp.full((16,), _gmax(tmax), jnp.float32)
    return _sc_aggregate(alpha128, xl, table, bst, gvec)


def kernel(x, edge_index, edge_attr, batch, Wl1, bl1, Wr1, br1, We1, be1,
           att1, bias1, Wl2, bl2, Wr2, br2, We2, be2, att2, bias2):
    src = edge_index[0]
    dst = edge_index[1]
    hist = _sc_hist(dst)
    table, bst = _sc_sort(src, dst, hist)
    e1 = _edge_project(edge_attr, We1, be1)
    e2 = _edge_project(edge_attr, We2, be2)
    h = _gat_layer(x, src, dst, e1, table, bst, Wl1, bl1, Wr1, br1, att1)
    bl2f = bias1 @ Wl2 + bl2
    br2f = bias1 @ Wr2 + br2
    h2 = _gat_layer(h, src, dst, e2, table, bst, Wl2, bl2f, Wr2, br2f, att2)
    return _mean_pool(h2, batch, bias2)


# speedup vs baseline: 7.6231x; 1.0353x over previous
import dataclasses
import functools

import jax
import jax.numpy as jnp
from jax import lax
from jax.experimental import pallas as pl
from jax.experimental.pallas import tpu as pltpu
from jax.experimental.pallas import tpu_sc as plsc

N = 10000
E = 160000
D = 512
H = 8
C = 64
HC = H * C
ED = 7
G = 64
NEG_SLOPE = 0.2


def _mm2_body(x_ref, wl_ref, wr_ref, bl_ref, br_ref, xl_ref, xr_ref):
    x = x_ref[...]
    xl_ref[...] = (
        jnp.dot(x, wl_ref[...], preferred_element_type=jnp.float32) + bl_ref[...]
    )
    xr_ref[...] = (
        jnp.dot(x, wr_ref[...], preferred_element_type=jnp.float32) + br_ref[...]
    )


def _dual_project(x, Wl, bl, Wr, br):
    nrows = x.shape[0]
    blk = 400
    grid = (nrows // blk,)
    return pl.pallas_call(
        _mm2_body,
        grid=grid,
        in_specs=[
            pl.BlockSpec((blk, D), lambda i: (i, 0)),
            pl.BlockSpec((D, HC), lambda i: (0, 0)),
            pl.BlockSpec((D, HC), lambda i: (0, 0)),
            pl.BlockSpec((1, HC), lambda i: (0, 0)),
            pl.BlockSpec((1, HC), lambda i: (0, 0)),
        ],
        out_specs=[
            pl.BlockSpec((blk, HC), lambda i: (i, 0)),
            pl.BlockSpec((blk, HC), lambda i: (i, 0)),
        ],
        out_shape=[
            jax.ShapeDtypeStruct((nrows, HC), jnp.float32),
            jax.ShapeDtypeStruct((nrows, HC), jnp.float32),
        ],
    )(x, Wl, Wr, bl.reshape(1, HC), br.reshape(1, HC))


def _edge_mm_body(a_ref, w_ref, b_ref, o_ref):
    o_ref[...] = (
        jnp.dot(a_ref[...], w_ref[...], preferred_element_type=jnp.float32)
        + b_ref[...]
    )


def _edge_project(edge_attr, We, be):
    a = jnp.pad(edge_attr, ((0, 0), (0, 8 - ED)))
    w = jnp.pad(We, ((0, 8 - ED), (0, 0)))
    blk = 2000
    return pl.pallas_call(
        _edge_mm_body,
        grid=(E // blk,),
        in_specs=[
            pl.BlockSpec((blk, 8), lambda i: (i, 0)),
            pl.BlockSpec((8, HC), lambda i: (0, 0)),
            pl.BlockSpec((1, HC), lambda i: (0, 0)),
        ],
        out_specs=pl.BlockSpec((blk, HC), lambda i: (i, 0)),
        out_shape=jax.ShapeDtypeStruct((E, HC), jnp.float32),
    )(a, w, be.reshape(1, HC))


def _pool_body(h_ref, oh_ref, sums_ref, cnt_ref):
    i = pl.program_id(0)

    @pl.when(i == 0)
    def _init():
        sums_ref[...] = jnp.zeros_like(sums_ref)
        cnt_ref[...] = jnp.zeros_like(cnt_ref)

    oh = oh_ref[...]
    sums_ref[...] += jnp.dot(
        oh.T, h_ref[...], preferred_element_type=jnp.float32
    )
    cnt_ref[...] += jnp.sum(oh, axis=0, keepdims=True)


def _mean_pool(h, batch, bias):
    blk = 400
    onehot = (batch[:, None] == jnp.arange(G)[None, :]).astype(jnp.float32)
    sums, cnt = pl.pallas_call(
        _pool_body,
        grid=(N // blk,),
        in_specs=[
            pl.BlockSpec((blk, HC), lambda i: (i, 0)),
            pl.BlockSpec((blk, G), lambda i: (i, 0)),
        ],
        out_specs=[
            pl.BlockSpec((G, HC), lambda i: (0, 0)),
            pl.BlockSpec((1, G), lambda i: (0, 0)),
        ],
        out_shape=[
            jax.ShapeDtypeStruct((G, HC), jnp.float32),
            jax.ShapeDtypeStruct((1, G), jnp.float32),
        ],
    )(h, onehot)
    return sums / jnp.maximum(cnt[0], 1.0)[:, None] + bias[None, :]


NWORK = 32
EPW = E // NWORK
ABLK = 40
NEG = -1e30

_SC_MESH = dict(core_axis_name="c", subcore_axis_name="s")

_SC_PARAMS = pltpu.CompilerParams()
if "needs_layout_passes" in pltpu.CompilerParams.__dataclass_fields__:
    _SC_PARAMS = dataclasses.replace(_SC_PARAMS, needs_layout_passes=False)


def _alpha_body(xl_hbm, xr_hbm, e_hbm, src_hbm, dst_hbm, att_hbm,
                alpha_hbm, tmax_hbm,
                sidx_v, didx_v, xl_v, xr_v, e_v, alpha_v, att_v, maxv_v, sem):
    wid = lax.axis_index("s") * 2 + lax.axis_index("c")
    base_t = wid * EPW
    pltpu.sync_copy(att_hbm, att_v)
    pltpu.sync_copy(src_hbm.at[pl.ds(base_t, EPW)], sidx_v)
    pltpu.sync_copy(dst_hbm.at[pl.ds(base_t, EPW)], didx_v)
    maxv_v[...] = jnp.full((16,), NEG, jnp.float32)

    @pl.loop(0, EPW // ABLK)
    def _blk(b):
        base = base_t + b * ABLK
        c1 = pltpu.async_copy(xl_hbm.at[sidx_v.at[pl.ds(b * ABLK, ABLK)]],
                              xl_v, sem)
        c2 = pltpu.async_copy(xr_hbm.at[didx_v.at[pl.ds(b * ABLK, ABLK)]],
                              xr_v, sem)
        c3 = pltpu.async_copy(e_hbm.at[pl.ds(base, ABLK)], e_v, sem)
        c1.wait()
        c2.wait()
        c3.wait()

        lane = lax.iota(jnp.int32, 16)

        @pl.loop(0, ABLK)
        def _edge(j):
            row = jnp.full((16,), NEG, jnp.float32)
            for h in range(H):
                acc = jnp.zeros((16,), jnp.float32)
                for q in range(C // 16):
                    sl = pl.ds(h * C + q * 16, 16)
                    m = xl_v[j, sl] + xr_v[j, sl] + e_v[j, sl]
                    m = jnp.where(m > 0, m, NEG_SLOPE * m)
                    acc = acc + m * att_v[h, pl.ds(q * 16, 16)]
                row = jnp.where(lane == h, jnp.sum(acc), row)
            alpha_v[j, pl.ds(0, 16)] = row

        @pl.loop(0, ABLK)
        def _mx(j):
            maxv_v[...] = jnp.maximum(maxv_v[...], alpha_v[j, pl.ds(0, 16)])

        pltpu.sync_copy(alpha_v, alpha_hbm.at[pl.ds(base, ABLK)])

    pltpu.sync_copy(maxv_v, tmax_hbm.at[wid])


def _sc_alpha(xl, xr, e, src, dst, att):
    kern = functools.partial(
        pl.kernel,
        mesh=plsc.VectorSubcoreMesh(**_SC_MESH),
        compiler_params=_SC_PARAMS,
        out_type=[
            jax.ShapeDtypeStruct((E, 128), jnp.float32),
            jax.ShapeDtypeStruct((NWORK, 16), jnp.float32),
        ],
        scratch_types=[
            pltpu.VMEM((EPW,), jnp.int32),
            pltpu.VMEM((EPW,), jnp.int32),
            pltpu.VMEM((ABLK, HC), jnp.float32),
            pltpu.VMEM((ABLK, HC), jnp.float32),
            pltpu.VMEM((ABLK, HC), jnp.float32),
            pltpu.VMEM((ABLK, 128), jnp.float32),
            pltpu.VMEM((H, C), jnp.float32),
            pltpu.VMEM((16,), jnp.float32),
            pltpu.SemaphoreType.DMA,
        ],
    )(_alpha_body)
    return kern(xl, xr, e, src, dst, att)


NB = 256
BN = 40
TBL = E + 16
SLC1 = 5008
SLC0 = E - (NWORK - 1) * SLC1
BL0 = SLC0 // 16
BL1 = SLC1 // 16


def _hist_body(dst_hbm, hist_hbm, dstb_v, hist_v):
    wid = lax.axis_index("s") * 2 + lax.axis_index("c")
    base = wid * SLC1
    pltpu.sync_copy(dst_hbm.at[pl.ds(base, SLC0)], dstb_v.at[pl.ds(0, SLC0)])

    @pl.when(wid < NWORK - 1)
    def _ld_tail():
        pltpu.sync_copy(dst_hbm.at[pl.ds(base + SLC0, SLC1 - SLC0)],
                        dstb_v.at[pl.ds(SLC0, SLC1 - SLC0)])

    lane = lax.iota(jnp.int32, 16)
    zi = jnp.zeros((16,), jnp.int32)
    for i in range(NB // 16):
        hist_v[pl.ds(i * 16, 16)] = zi

    def _blk(b):
        bv = dstb_v[pl.ds(b * 16, 16)] // BN
        for l in range(16):
            b_l = bv[l]
            fb = (b_l // 16) * 16
            w = hist_v[pl.ds(fb, 16)]
            hist_v[pl.ds(fb, 16)] = w + jnp.where(lane == b_l - fb, 1, 0)

    pl.loop(0, BL0)(_blk)

    @pl.when(wid < NWORK - 1)
    def _tail():
        pl.loop(BL0, BL1)(_blk)

    pltpu.sync_copy(hist_v, hist_hbm.at[wid])


def _sc_hist(dst):
    kern = functools.partial(
        pl.kernel,
        mesh=plsc.VectorSubcoreMesh(**_SC_MESH),
        compiler_params=_SC_PARAMS,
        out_type=jax.ShapeDtypeStruct((NWORK, NB), jnp.int32),
        scratch_types=[
            pltpu.VMEM((SLC1,), jnp.int32),
            pltpu.VMEM((NB,), jnp.int32),
        ],
    )(_hist_body)
    return kern(dst)


def _sort_body(src_hbm, dst_hbm, hist_hbm, table_hbm, bst_hbm,
               srcb_v, dstb_v, h32_v, offs_v, trip_v, pos_v, bst_v, sem):
    wid = lax.axis_index("s") * 2 + lax.axis_index("c")
    base = wid * SLC1
    pltpu.sync_copy(src_hbm.at[pl.ds(base, SLC0)], srcb_v.at[pl.ds(0, SLC0)])
    pltpu.sync_copy(dst_hbm.at[pl.ds(base, SLC0)], dstb_v.at[pl.ds(0, SLC0)])

    @pl.when(wid < NWORK - 1)
    def _ld_tail():
        pltpu.sync_copy(src_hbm.at[pl.ds(base + SLC0, SLC1 - SLC0)],
                        srcb_v.at[pl.ds(SLC0, SLC1 - SLC0)])
        pltpu.sync_copy(dst_hbm.at[pl.ds(base + SLC0, SLC1 - SLC0)],
                        dstb_v.at[pl.ds(SLC0, SLC1 - SLC0)])

    pltpu.sync_copy(hist_hbm, h32_v)
    lane = lax.iota(jnp.int32, 16)

    run = jnp.int32(0)
    for i in range(NB // 16):
        tot = jnp.zeros((16,), jnp.int32)
        pre = jnp.zeros((16,), jnp.int32)
        for w in range(NWORK):
            row = h32_v[w, pl.ds(i * 16, 16)]
            tot = tot + row
            pre = pre + jnp.where(wid > w, row, 0)
        cs = plsc.cumsum(tot)
        excl = cs - tot
        offs_v[pl.ds(i * 16, 16)] = run + excl + pre
        startv = run + excl
        endv = run + cs
        for l in range(16):
            bst_v[i * 16 + l, pl.ds(0, 16)] = (
                jnp.where(lane == 0, startv[l], 0)
                + jnp.where(lane == 1, endv[l], 0)
            )
        run = run + cs[15]

    @pl.when(wid == 0)
    def _aux():
        pltpu.sync_copy(bst_v, bst_hbm)

        @pl.loop(0, 16)
        def _sr(j):
            trip_v[j, pl.ds(0, 16)] = jnp.where(lane == 1, -1, 0)

        pltpu.sync_copy(trip_v, table_hbm.at[pl.ds(E, 16)])

    def _blk(b):
        dv = dstb_v[pl.ds(b * 16, 16)]
        sv = srcb_v[pl.ds(b * 16, 16)]
        bv = dv // BN
        eidv = base + b * 16 + lane
        posacc = jnp.zeros((16,), jnp.int32)
        for l in range(16):
            b_l = bv[l]
            fb = (b_l // 16) * 16
            w = offs_v[pl.ds(fb, 16)]
            sel = lane == (b_l - fb)
            pos_l = jnp.max(jnp.where(sel, w, -1))
            offs_v[pl.ds(fb, 16)] = w + jnp.where(sel, 1, 0)
            posacc = jnp.where(lane == l, pos_l, posacc)
            trip_v[l, pl.ds(0, 16)] = (
                jnp.where(lane == 0, sv[l], 0)
                + jnp.where(lane == 1, dv[l], 0)
                + jnp.where(lane == 2, eidv[l], 0)
            )
        pos_v[0, pl.ds(0, 16)] = posacc
        pltpu.async_copy(trip_v, table_hbm.at[pos_v.at[0]], sem).wait()

    pl.loop(0, BL0)(_blk)

    @pl.when(wid < NWORK - 1)
    def _tail():
        pl.loop(BL0, BL1)(_blk)


def _sc_sort(src, dst, hist):
    kern = functools.partial(
        pl.kernel,
        mesh=plsc.VectorSubcoreMesh(**_SC_MESH),
        compiler_params=_SC_PARAMS,
        out_type=[
            jax.ShapeDtypeStruct((TBL, 128), jnp.int32),
            jax.ShapeDtypeStruct((NB, 16), jnp.int32),
        ],
        scratch_types=[
            pltpu.VMEM((SLC1,), jnp.int32),
            pltpu.VMEM((SLC1,), jnp.int32),
            pltpu.VMEM((NWORK, NB), jnp.int32),
            pltpu.VMEM((NB,), jnp.int32),
            pltpu.VMEM((16, 128), jnp.int32),
            pltpu.VMEM((1, 16), jnp.int32),
            pltpu.VMEM((NB, 16), jnp.int32),
            pltpu.SemaphoreType.DMA,
        ],
    )(_sort_body)
    return kern(src, dst, hist)


def _agg_body(alpha_hbm, xl_hbm, table_hbm, bst_hbm, tmax_hbm, out_hbm,
              bst_v, trip_v, eix_v, six_v, axr_v, xlr_v, acc_v, den_v,
              tmax_v, sem):
    cid = lax.axis_index("c")
    sid = lax.axis_index("s")
    pltpu.sync_copy(tmax_hbm, tmax_v)
    pltpu.sync_copy(bst_hbm, bst_v)
    mrow = jnp.full((16,), NEG, jnp.float32)
    for w in range(NWORK):
        mrow = jnp.maximum(mrow, tmax_v[w, pl.ds(0, 16)])
    gvreg = jnp.zeros((16,), jnp.float32) + jnp.max(mrow)
    lane = lax.iota(jnp.int32, 16)
    zf = jnp.zeros((16,), jnp.float32)

    @pl.loop(0, NB // NWORK)
    def _bk(k):
        b = (cid * (NB // NWORK) + k) * 16 + sid
        brow = bst_v[b, pl.ds(0, 16)]
        start = brow[0]
        end = brow[1]
        lo = b * BN

        @pl.loop(0, BN)
        def _z(r):
            for q in range(HC // 16):
                acc_v[r, pl.ds(q * 16, 16)] = zf
            den_v[r, pl.ds(0, 16)] = zf

        a0 = (start // 16) * 16

        @pl.loop(0, (end - a0 + 15) // 16)
        def _blk(g):
            pltpu.sync_copy(table_hbm.at[pl.ds(a0 + g * 16, 16)], trip_v)
            ei = jnp.zeros((16,), jnp.int32)
            si = jnp.zeros((16,), jnp.int32)
            for l in range(16):
                tr = trip_v[l, pl.ds(0, 16)]
                ei = jnp.where(lane == l, tr[2], ei)
                si = jnp.where(lane == l, tr[0], si)
            eix_v[...] = ei
            six_v[...] = si
            c1 = pltpu.async_copy(alpha_hbm.at[eix_v], axr_v, sem)
            c2 = pltpu.async_copy(xl_hbm.at[six_v], xlr_v, sem)
            c1.wait()
            c2.wait()

            @pl.loop(0, 16)
            def _edge(j):
                dl = trip_v[j, pl.ds(0, 16)][1] - lo

                @pl.when((dl >= 0) & (dl < BN))
                def _proc():
                    ex = jnp.exp(axr_v[j, pl.ds(0, 16)] - gvreg)
                    den_v[dl, pl.ds(0, 16)] = den_v[dl, pl.ds(0, 16)] + ex
                    for h in range(H):
                        a_s = ex[h]
                        for q in range(C // 16):
                            sl = pl.ds(h * C + q * 16, 16)
                            acc_v[dl, sl] = (acc_v[dl, sl]
                                             + xlr_v[j, sl] * a_s)

        @pl.when(lo < N)
        def _dump():
            @pl.loop(0, BN)
            def _n(r):
                invv = 1.0 / (den_v[r, pl.ds(0, 16)] + 1e-16)
                for h in range(H):
                    iv = invv[h]
                    for q in range(C // 16):
                        sl = pl.ds(h * C + q * 16, 16)
                        acc_v[r, sl] = acc_v[r, sl] * iv

            pltpu.sync_copy(acc_v, out_hbm.at[pl.ds(lo, BN)])


def _sc_aggregate(alpha128, xl, table, bst, tmax):
    kern = functools.partial(
        pl.kernel,
        mesh=plsc.VectorSubcoreMesh(**_SC_MESH),
        compiler_params=_SC_PARAMS,
        out_type=jax.ShapeDtypeStruct((N, HC), jnp.float32),
        scratch_types=[
            pltpu.VMEM((NB, 16), jnp.int32),
            pltpu.VMEM((16, 128), jnp.int32),
            pltpu.VMEM((16,), jnp.int32),
            pltpu.VMEM((16,), jnp.int32),
            pltpu.VMEM((16, 128), jnp.float32),
            pltpu.VMEM((16, HC), jnp.float32),
            pltpu.VMEM((BN, HC), jnp.float32),
            pltpu.VMEM((BN, 16), jnp.float32),
            pltpu.VMEM((NWORK, 16), jnp.float32),
            pltpu.SemaphoreType.DMA,
        ],
    )(_agg_body)
    return kern(alpha128, xl, table, bst, tmax)


def _gmax_body(t_ref, o_ref):
    o_ref[...] = jnp.max(t_ref[...]).reshape(1, 1)


def _gmax(tmax):
    return pl.pallas_call(
        _gmax_body,
        out_shape=jax.ShapeDtypeStruct((1, 1), jnp.float32),
    )(tmax)[0, 0]


def _gat_layer(x, src, dst, e, table, bst, Wl, bl, Wr, br, att):
    xl, xr = _dual_project(x, Wl, bl, Wr, br)
    alpha128, tmax = _sc_alpha(xl, xr, e, src, dst, att)
    return _sc_aggregate(alpha128, xl, table, bst, tmax)


def kernel(x, edge_index, edge_attr, batch, Wl1, bl1, Wr1, br1, We1, be1,
           att1, bias1, Wl2, bl2, Wr2, br2, We2, be2, att2, bias2):
    src = edge_index[0]
    dst = edge_index[1]
    hist = _sc_hist(dst)
    table, bst = _sc_sort(src, dst, hist)
    e1 = _edge_project(edge_attr, We1, be1)
    e2 = _edge_project(edge_attr, We2, be2)
    h = _gat_layer(x, src, dst, e1, table, bst, Wl1, bl1, Wr1, br1, att1)
    bl2f = bias1 @ Wl2 + bl2
    br2f = bias1 @ Wr2 + br2
    h2 = _gat_layer(h, src, dst, e2, table, bst, Wl2, bl2f, Wr2, br2f, att2)
    return _mean_pool(h2, batch, bias2)


# speedup vs baseline: 7.6931x; 1.0092x over previous
import dataclasses
import functools

import jax
import jax.numpy as jnp
from jax import lax
from jax.experimental import pallas as pl
from jax.experimental.pallas import tpu as pltpu
from jax.experimental.pallas import tpu_sc as plsc

N = 10000
E = 160000
D = 512
H = 8
C = 64
HC = H * C
ED = 7
G = 64
NEG_SLOPE = 0.2


def _mm2_body(x_ref, wl_ref, wr_ref, bl_ref, br_ref, xl_ref, xr_ref):
    x = x_ref[...]
    xl_ref[...] = (
        jnp.dot(x, wl_ref[...], preferred_element_type=jnp.float32) + bl_ref[...]
    )
    xr_ref[...] = (
        jnp.dot(x, wr_ref[...], preferred_element_type=jnp.float32) + br_ref[...]
    )


def _dual_project(x, Wl, bl, Wr, br):
    nrows = x.shape[0]
    blk = 400
    grid = (nrows // blk,)
    return pl.pallas_call(
        _mm2_body,
        grid=grid,
        in_specs=[
            pl.BlockSpec((blk, D), lambda i: (i, 0)),
            pl.BlockSpec((D, HC), lambda i: (0, 0)),
            pl.BlockSpec((D, HC), lambda i: (0, 0)),
            pl.BlockSpec((1, HC), lambda i: (0, 0)),
            pl.BlockSpec((1, HC), lambda i: (0, 0)),
        ],
        out_specs=[
            pl.BlockSpec((blk, HC), lambda i: (i, 0)),
            pl.BlockSpec((blk, HC), lambda i: (i, 0)),
        ],
        out_shape=[
            jax.ShapeDtypeStruct((nrows, HC), jnp.float32),
            jax.ShapeDtypeStruct((nrows, HC), jnp.float32),
        ],
    )(x, Wl, Wr, bl.reshape(1, HC), br.reshape(1, HC))


def _edge_mm_body(a_ref, w_ref, b_ref, o_ref):
    o_ref[...] = (
        jnp.dot(a_ref[...], w_ref[...], preferred_element_type=jnp.float32)
        + b_ref[...]
    )


def _edge_project(edge_attr, We, be):
    a = jnp.pad(edge_attr, ((0, 0), (0, 8 - ED)))
    w = jnp.pad(We, ((0, 8 - ED), (0, 0)))
    blk = 2000
    return pl.pallas_call(
        _edge_mm_body,
        grid=(E // blk,),
        in_specs=[
            pl.BlockSpec((blk, 8), lambda i: (i, 0)),
            pl.BlockSpec((8, HC), lambda i: (0, 0)),
            pl.BlockSpec((1, HC), lambda i: (0, 0)),
        ],
        out_specs=pl.BlockSpec((blk, HC), lambda i: (i, 0)),
        out_shape=jax.ShapeDtypeStruct((E + blk, HC), jnp.float32),
    )(a, w, be.reshape(1, HC))


def _pool_body(h_ref, oh_ref, sums_ref, cnt_ref):
    i = pl.program_id(0)

    @pl.when(i == 0)
    def _init():
        sums_ref[...] = jnp.zeros_like(sums_ref)
        cnt_ref[...] = jnp.zeros_like(cnt_ref)

    oh = oh_ref[...]
    sums_ref[...] += jnp.dot(
        oh.T, h_ref[...], preferred_element_type=jnp.float32
    )
    cnt_ref[...] += jnp.sum(oh, axis=0, keepdims=True)


def _mean_pool(h, batch, bias):
    blk = 400
    onehot = (batch[:, None] == jnp.arange(G)[None, :]).astype(jnp.float32)
    sums, cnt = pl.pallas_call(
        _pool_body,
        grid=(N // blk,),
        in_specs=[
            pl.BlockSpec((blk, HC), lambda i: (i, 0)),
            pl.BlockSpec((blk, G), lambda i: (i, 0)),
        ],
        out_specs=[
            pl.BlockSpec((G, HC), lambda i: (0, 0)),
            pl.BlockSpec((1, G), lambda i: (0, 0)),
        ],
        out_shape=[
            jax.ShapeDtypeStruct((G, HC), jnp.float32),
            jax.ShapeDtypeStruct((1, G), jnp.float32),
        ],
    )(h, onehot)
    return sums / jnp.maximum(cnt[0], 1.0)[:, None] + bias[None, :]


NWORK = 32
EPW = E // NWORK
ABLK = 40
NEG = -1e30

_SC_MESH = dict(core_axis_name="c", subcore_axis_name="s")

_SC_PARAMS = pltpu.CompilerParams()
if "needs_layout_passes" in pltpu.CompilerParams.__dataclass_fields__:
    _SC_PARAMS = dataclasses.replace(_SC_PARAMS, needs_layout_passes=False)


def _alpha_body(xl_hbm, xr_hbm, e_hbm, src_hbm, dst_hbm, att_hbm,
                alpha_hbm, tmax_hbm,
                sidx_v, didx_v, xl_v0, xr_v0, xl_v1, xr_v1, e_v0,
                alpha_v, att_v, maxv_v, semA, semB):
    wid = lax.axis_index("s") * 2 + lax.axis_index("c")
    base_t = wid * EPW
    pltpu.sync_copy(att_hbm, att_v)
    pltpu.sync_copy(src_hbm.at[pl.ds(base_t, EPW)],
                    sidx_v.at[pl.ds(0, EPW)])
    pltpu.sync_copy(dst_hbm.at[pl.ds(base_t, EPW)],
                    didx_v.at[pl.ds(0, EPW)])
    sidx_v[pl.ds(EPW, 16)] = jnp.zeros((16,), jnp.int32)
    sidx_v[pl.ds(EPW + 16, 16)] = jnp.zeros((16,), jnp.int32)
    didx_v[pl.ds(EPW, 16)] = jnp.zeros((16,), jnp.int32)
    didx_v[pl.ds(EPW + 16, 16)] = jnp.zeros((16,), jnp.int32)
    maxv_v[...] = jnp.full((16,), NEG, jnp.float32)
    lane = lax.iota(jnp.int32, 16)

    def issue(b, xl_v, xr_v, sem):
        pltpu.async_copy(xl_hbm.at[sidx_v.at[pl.ds(b * ABLK, ABLK)]],
                         xl_v, sem)
        pltpu.async_copy(xr_hbm.at[didx_v.at[pl.ds(b * ABLK, ABLK)]],
                         xr_v, sem)

    def wait(xl_v, xr_v, sem):
        pltpu.make_async_copy(e_hbm.at[pl.ds(0, ABLK)], xl_v, sem).wait()
        pltpu.make_async_copy(e_hbm.at[pl.ds(0, ABLK)], xr_v, sem).wait()

    def compute(b, xl_v, xr_v):
        pltpu.sync_copy(e_hbm.at[pl.ds(base_t + b * ABLK, ABLK)], e_v0)
        e_v = e_v0
        @pl.loop(0, ABLK)
        def _edge(j):
            row = jnp.full((16,), NEG, jnp.float32)
            for h in range(H):
                acc = jnp.zeros((16,), jnp.float32)
                for q in range(C // 16):
                    sl = pl.ds(h * C + q * 16, 16)
                    m = xl_v[j, sl] + xr_v[j, sl] + e_v[j, sl]
                    m = jnp.where(m > 0, m, NEG_SLOPE * m)
                    acc = acc + m * att_v[h, pl.ds(q * 16, 16)]
                row = jnp.where(lane == h, jnp.sum(acc), row)
            alpha_v[j, pl.ds(0, 16)] = row

        @pl.loop(0, ABLK)
        def _mx(j):
            maxv_v[...] = jnp.maximum(maxv_v[...], alpha_v[j, pl.ds(0, 16)])

        pltpu.sync_copy(alpha_v, alpha_hbm.at[pl.ds(base_t + b * ABLK, ABLK)])

    issue(0, xl_v0, xr_v0, semA)

    @pl.loop(0, EPW // ABLK // 2)
    def _pair(p):
        issue(2 * p + 1, xl_v1, xr_v1, semB)
        wait(xl_v0, xr_v0, semA)
        compute(2 * p, xl_v0, xr_v0)
        issue(2 * p + 2, xl_v0, xr_v0, semA)
        wait(xl_v1, xr_v1, semB)
        compute(2 * p + 1, xl_v1, xr_v1)

    wait(xl_v0, xr_v0, semA)
    compute(EPW // ABLK - 1, xl_v0, xr_v0)
    pltpu.sync_copy(maxv_v, tmax_hbm.at[wid])


def _sc_alpha(xl, xr, e, src, dst, att):
    kern = functools.partial(
        pl.kernel,
        mesh=plsc.VectorSubcoreMesh(**_SC_MESH),
        compiler_params=_SC_PARAMS,
        out_type=[
            jax.ShapeDtypeStruct((E, 128), jnp.float32),
            jax.ShapeDtypeStruct((NWORK, 16), jnp.float32),
        ],
        scratch_types=[
            pltpu.VMEM((EPW + 2 * ABLK,), jnp.int32),
            pltpu.VMEM((EPW + 2 * ABLK,), jnp.int32),
            pltpu.VMEM((ABLK, HC), jnp.float32),
            pltpu.VMEM((ABLK, HC), jnp.float32),
            pltpu.VMEM((ABLK, HC), jnp.float32),
            pltpu.VMEM((ABLK, HC), jnp.float32),
            pltpu.VMEM((ABLK, HC), jnp.float32),
            pltpu.VMEM((ABLK, 128), jnp.float32),
            pltpu.VMEM((H, C), jnp.float32),
            pltpu.VMEM((16,), jnp.float32),
            pltpu.SemaphoreType.DMA,
            pltpu.SemaphoreType.DMA,
        ],
    )(_alpha_body)
    return kern(xl, xr, e, src, dst, att)


NB = 256
BN = 40
TBL = E + 16
SLC1 = 5008
SLC0 = E - (NWORK - 1) * SLC1
BL0 = SLC0 // 16
BL1 = SLC1 // 16


def _hist_body(dst_hbm, hist_hbm, dstb_v, hist_v):
    wid = lax.axis_index("s") * 2 + lax.axis_index("c")
    base = wid * SLC1
    pltpu.sync_copy(dst_hbm.at[pl.ds(base, SLC0)], dstb_v.at[pl.ds(0, SLC0)])

    @pl.when(wid < NWORK - 1)
    def _ld_tail():
        pltpu.sync_copy(dst_hbm.at[pl.ds(base + SLC0, SLC1 - SLC0)],
                        dstb_v.at[pl.ds(SLC0, SLC1 - SLC0)])

    lane = lax.iota(jnp.int32, 16)
    zi = jnp.zeros((16,), jnp.int32)
    for i in range(NB // 16):
        hist_v[pl.ds(i * 16, 16)] = zi

    def _blk(b):
        bv = dstb_v[pl.ds(b * 16, 16)] // BN
        for l in range(16):
            b_l = bv[l]
            fb = (b_l // 16) * 16
            w = hist_v[pl.ds(fb, 16)]
            hist_v[pl.ds(fb, 16)] = w + jnp.where(lane == b_l - fb, 1, 0)

    pl.loop(0, BL0)(_blk)

    @pl.when(wid < NWORK - 1)
    def _tail():
        pl.loop(BL0, BL1)(_blk)

    pltpu.sync_copy(hist_v, hist_hbm.at[wid])


def _sc_hist(dst):
    kern = functools.partial(
        pl.kernel,
        mesh=plsc.VectorSubcoreMesh(**_SC_MESH),
        compiler_params=_SC_PARAMS,
        out_type=jax.ShapeDtypeStruct((NWORK, NB), jnp.int32),
        scratch_types=[
            pltpu.VMEM((SLC1,), jnp.int32),
            pltpu.VMEM((NB,), jnp.int32),
        ],
    )(_hist_body)
    return kern(dst)


def _sort_body(src_hbm, dst_hbm, hist_hbm, table_hbm, bst_hbm,
               srcb_v, dstb_v, h32_v, offs_v, trip_v, pos_v, bst_v, sem):
    wid = lax.axis_index("s") * 2 + lax.axis_index("c")
    base = wid * SLC1
    pltpu.sync_copy(src_hbm.at[pl.ds(base, SLC0)], srcb_v.at[pl.ds(0, SLC0)])
    pltpu.sync_copy(dst_hbm.at[pl.ds(base, SLC0)], dstb_v.at[pl.ds(0, SLC0)])

    @pl.when(wid < NWORK - 1)
    def _ld_tail():
        pltpu.sync_copy(src_hbm.at[pl.ds(base + SLC0, SLC1 - SLC0)],
                        srcb_v.at[pl.ds(SLC0, SLC1 - SLC0)])
        pltpu.sync_copy(dst_hbm.at[pl.ds(base + SLC0, SLC1 - SLC0)],
                        dstb_v.at[pl.ds(SLC0, SLC1 - SLC0)])

    pltpu.sync_copy(hist_hbm, h32_v)
    lane = lax.iota(jnp.int32, 16)

    run = jnp.int32(0)
    for i in range(NB // 16):
        tot = jnp.zeros((16,), jnp.int32)
        pre = jnp.zeros((16,), jnp.int32)
        for w in range(NWORK):
            row = h32_v[w, pl.ds(i * 16, 16)]
            tot = tot + row
            pre = pre + jnp.where(wid > w, row, 0)
        cs = plsc.cumsum(tot)
        excl = cs - tot
        offs_v[pl.ds(i * 16, 16)] = run + excl + pre
        startv = run + excl
        endv = run + cs
        for l in range(16):
            bst_v[i * 16 + l, pl.ds(0, 16)] = (
                jnp.where(lane == 0, startv[l], 0)
                + jnp.where(lane == 1, endv[l], 0)
            )
        run = run + cs[15]

    @pl.when(wid == 0)
    def _aux():
        pltpu.sync_copy(bst_v, bst_hbm)

        @pl.loop(0, 16)
        def _sr(j):
            trip_v[j, pl.ds(0, 16)] = jnp.where(lane == 1, -1, 0)

        pltpu.sync_copy(trip_v, table_hbm.at[pl.ds(E, 16)])

    def _blk(b):
        dv = dstb_v[pl.ds(b * 16, 16)]
        sv = srcb_v[pl.ds(b * 16, 16)]
        bv = dv // BN
        eidv = base + b * 16 + lane
        posacc = jnp.zeros((16,), jnp.int32)
        for l in range(16):
            b_l = bv[l]
            fb = (b_l // 16) * 16
            w = offs_v[pl.ds(fb, 16)]
            sel = lane == (b_l - fb)
            pos_l = jnp.max(jnp.where(sel, w, -1))
            offs_v[pl.ds(fb, 16)] = w + jnp.where(sel, 1, 0)
            posacc = jnp.where(lane == l, pos_l, posacc)
            trip_v[l, pl.ds(0, 16)] = (
                jnp.where(lane == 0, sv[l], 0)
                + jnp.where(lane == 1, dv[l], 0)
                + jnp.where(lane == 2, eidv[l], 0)
            )
        pos_v[0, pl.ds(0, 16)] = posacc
        pltpu.async_copy(trip_v, table_hbm.at[pos_v.at[0]], sem).wait()

    pl.loop(0, BL0)(_blk)

    @pl.when(wid < NWORK - 1)
    def _tail():
        pl.loop(BL0, BL1)(_blk)


def _sc_sort(src, dst, hist):
    kern = functools.partial(
        pl.kernel,
        mesh=plsc.VectorSubcoreMesh(**_SC_MESH),
        compiler_params=_SC_PARAMS,
        out_type=[
            jax.ShapeDtypeStruct((TBL, 128), jnp.int32),
            jax.ShapeDtypeStruct((NB, 16), jnp.int32),
        ],
        scratch_types=[
            pltpu.VMEM((SLC1,), jnp.int32),
            pltpu.VMEM((SLC1,), jnp.int32),
            pltpu.VMEM((NWORK, NB), jnp.int32),
            pltpu.VMEM((NB,), jnp.int32),
            pltpu.VMEM((16, 128), jnp.int32),
            pltpu.VMEM((1, 16), jnp.int32),
            pltpu.VMEM((NB, 16), jnp.int32),
            pltpu.SemaphoreType.DMA,
        ],
    )(_sort_body)
    return kern(src, dst, hist)


def _agg_body(alpha_hbm, xl_hbm, table_hbm, bst_hbm, tmax_hbm, out_hbm,
              bst_v, trip_v, eix_v, six_v, axr_v, xlr_v, acc_v, den_v,
              tmax_v, sem):
    cid = lax.axis_index("c")
    sid = lax.axis_index("s")
    pltpu.sync_copy(tmax_hbm, tmax_v)
    pltpu.sync_copy(bst_hbm, bst_v)
    mrow = jnp.full((16,), NEG, jnp.float32)
    for w in range(NWORK):
        mrow = jnp.maximum(mrow, tmax_v[w, pl.ds(0, 16)])
    gvreg = jnp.zeros((16,), jnp.float32) + jnp.max(mrow)
    lane = lax.iota(jnp.int32, 16)
    zf = jnp.zeros((16,), jnp.float32)

    @pl.loop(0, NB // NWORK)
    def _bk(k):
        b = (cid * (NB // NWORK) + k) * 16 + sid
        brow = bst_v[b, pl.ds(0, 16)]
        start = brow[0]
        end = brow[1]
        lo = b * BN

        @pl.loop(0, BN)
        def _z(r):
            for q in range(HC // 16):
                acc_v[r, pl.ds(q * 16, 16)] = zf
            den_v[r, pl.ds(0, 16)] = zf

        a0 = (start // 16) * 16

        @pl.loop(0, (end - a0 + 15) // 16)
        def _blk(g):
            pltpu.sync_copy(table_hbm.at[pl.ds(a0 + g * 16, 16)], trip_v)
            ei = jnp.zeros((16,), jnp.int32)
            si = jnp.zeros((16,), jnp.int32)
            for l in range(16):
                tr = trip_v[l, pl.ds(0, 16)]
                ei = jnp.where(lane == l, tr[2], ei)
                si = jnp.where(lane == l, tr[0], si)
            eix_v[...] = ei
            six_v[...] = si
            c1 = pltpu.async_copy(alpha_hbm.at[eix_v], axr_v, sem)
            c2 = pltpu.async_copy(xl_hbm.at[six_v], xlr_v, sem)
            c1.wait()
            c2.wait()

            @pl.loop(0, 16)
            def _edge(j):
                dl = trip_v[j, pl.ds(0, 16)][1] - lo

                @pl.when((dl >= 0) & (dl < BN))
                def _proc():
                    ex = jnp.exp(axr_v[j, pl.ds(0, 16)] - gvreg)
                    den_v[dl, pl.ds(0, 16)] = den_v[dl, pl.ds(0, 16)] + ex
                    for h in range(H):
                        a_s = ex[h]
                        for q in range(C // 16):
                            sl = pl.ds(h * C + q * 16, 16)
                            acc_v[dl, sl] = (acc_v[dl, sl]
                                             + xlr_v[j, sl] * a_s)

        @pl.when(lo < N)
        def _dump():
            @pl.loop(0, BN)
            def _n(r):
                invv = 1.0 / (den_v[r, pl.ds(0, 16)] + 1e-16)
                for h in range(H):
                    iv = invv[h]
                    for q in range(C // 16):
                        sl = pl.ds(h * C + q * 16, 16)
                        acc_v[r, sl] = acc_v[r, sl] * iv

            pltpu.sync_copy(acc_v, out_hbm.at[pl.ds(lo, BN)])


def _sc_aggregate(alpha128, xl, table, bst, tmax):
    kern = functools.partial(
        pl.kernel,
        mesh=plsc.VectorSubcoreMesh(**_SC_MESH),
        compiler_params=_SC_PARAMS,
        out_type=jax.ShapeDtypeStruct((N, HC), jnp.float32),
        scratch_types=[
            pltpu.VMEM((NB, 16), jnp.int32),
            pltpu.VMEM((16, 128), jnp.int32),
            pltpu.VMEM((16,), jnp.int32),
            pltpu.VMEM((16,), jnp.int32),
            pltpu.VMEM((16, 128), jnp.float32),
            pltpu.VMEM((16, HC), jnp.float32),
            pltpu.VMEM((BN, HC), jnp.float32),
            pltpu.VMEM((BN, 16), jnp.float32),
            pltpu.VMEM((NWORK, 16), jnp.float32),
            pltpu.SemaphoreType.DMA,
        ],
    )(_agg_body)
    return kern(alpha128, xl, table, bst, tmax)


def _gmax_body(t_ref, o_ref):
    o_ref[...] = jnp.max(t_ref[...]).reshape(1, 1)


def _gmax(tmax):
    return pl.pallas_call(
        _gmax_body,
        out_shape=jax.ShapeDtypeStruct((1, 1), jnp.float32),
    )(tmax)[0, 0]


def _gat_layer(x, src, dst, e, table, bst, Wl, bl, Wr, br, att):
    xl, xr = _dual_project(x, Wl, bl, Wr, br)
    alpha128, tmax = _sc_alpha(xl, xr, e, src, dst, att)
    return _sc_aggregate(alpha128, xl, table, bst, tmax)


def kernel(x, edge_index, edge_attr, batch, Wl1, bl1, Wr1, br1, We1, be1,
           att1, bias1, Wl2, bl2, Wr2, br2, We2, be2, att2, bias2):
    src = edge_index[0]
    dst = edge_index[1]
    hist = _sc_hist(dst)
    table, bst = _sc_sort(src, dst, hist)
    e1 = _edge_project(edge_attr, We1, be1)
    e2 = _edge_project(edge_attr, We2, be2)
    h = _gat_layer(x, src, dst, e1, table, bst, Wl1, bl1, Wr1, br1, att1)
    bl2f = bias1 @ Wl2 + bl2
    br2f = bias1 @ Wr2 + br2
    h2 = _gat_layer(h, src, dst, e2, table, bst, Wl2, bl2f, Wr2, br2f, att2)
    return _mean_pool(h2, batch, bias2)


# speedup vs baseline: 7.7155x; 1.0029x over previous
import dataclasses
import functools

import jax
import jax.numpy as jnp
from jax import lax
from jax.experimental import pallas as pl
from jax.experimental.pallas import tpu as pltpu
from jax.experimental.pallas import tpu_sc as plsc

N = 10000
E = 160000
D = 512
H = 8
C = 64
HC = H * C
ED = 7
G = 64
NEG_SLOPE = 0.2


def _mm2_body(x_ref, wl_ref, wr_ref, bl_ref, br_ref, xl_ref, xr_ref):
    x = x_ref[...]
    xl_ref[...] = (
        jnp.dot(x, wl_ref[...], preferred_element_type=jnp.float32) + bl_ref[...]
    )
    xr_ref[...] = (
        jnp.dot(x, wr_ref[...], preferred_element_type=jnp.float32) + br_ref[...]
    )


def _dual_project(x, Wl, bl, Wr, br):
    nrows = x.shape[0]
    blk = 400
    grid = (nrows // blk,)
    return pl.pallas_call(
        _mm2_body,
        grid=grid,
        in_specs=[
            pl.BlockSpec((blk, D), lambda i: (i, 0)),
            pl.BlockSpec((D, HC), lambda i: (0, 0)),
            pl.BlockSpec((D, HC), lambda i: (0, 0)),
            pl.BlockSpec((1, HC), lambda i: (0, 0)),
            pl.BlockSpec((1, HC), lambda i: (0, 0)),
        ],
        out_specs=[
            pl.BlockSpec((blk, HC), lambda i: (i, 0)),
            pl.BlockSpec((blk, HC), lambda i: (i, 0)),
        ],
        out_shape=[
            jax.ShapeDtypeStruct((nrows, HC), jnp.float32),
            jax.ShapeDtypeStruct((nrows, HC), jnp.float32),
        ],
    )(x, Wl, Wr, bl.reshape(1, HC), br.reshape(1, HC))


def _edge_mm_body(a_ref, w_ref, b_ref, o_ref):
    o_ref[...] = (
        jnp.dot(a_ref[...], w_ref[...], preferred_element_type=jnp.float32)
        + b_ref[...]
    )


def _edge_project(edge_attr, We, be):
    a = jnp.pad(edge_attr, ((0, 0), (0, 8 - ED)))
    w = jnp.pad(We, ((0, 8 - ED), (0, 0)))
    blk = 2000
    return pl.pallas_call(
        _edge_mm_body,
        grid=(E // blk,),
        in_specs=[
            pl.BlockSpec((blk, 8), lambda i: (i, 0)),
            pl.BlockSpec((8, HC), lambda i: (0, 0)),
            pl.BlockSpec((1, HC), lambda i: (0, 0)),
        ],
        out_specs=pl.BlockSpec((blk, HC), lambda i: (i, 0)),
        out_shape=jax.ShapeDtypeStruct((E + blk, HC), jnp.float32),
    )(a, w, be.reshape(1, HC))


def _pool_body(h_ref, oh_ref, b_ref, out_ref, cnt_ref):
    i = pl.program_id(0)

    @pl.when(i == 0)
    def _init():
        out_ref[...] = jnp.zeros_like(out_ref)
        cnt_ref[...] = jnp.zeros_like(cnt_ref)

    oh = oh_ref[...]
    out_ref[...] += jnp.dot(
        oh.T, h_ref[...], preferred_element_type=jnp.float32
    )
    cnt_ref[...] += jnp.sum(oh, axis=0, keepdims=True)

    @pl.when(i == N // 400 - 1)
    def _fin():
        c = jnp.maximum(cnt_ref[...], 1.0)
        out_ref[...] = out_ref[...] / c.reshape(G, 1) + b_ref[...]


def _mean_pool(h, batch, bias):
    blk = 400
    onehot = (batch[:, None] == jnp.arange(G)[None, :]).astype(jnp.float32)
    out, _ = pl.pallas_call(
        _pool_body,
        grid=(N // blk,),
        in_specs=[
            pl.BlockSpec((blk, HC), lambda i: (i, 0)),
            pl.BlockSpec((blk, G), lambda i: (i, 0)),
            pl.BlockSpec((1, HC), lambda i: (0, 0)),
        ],
        out_specs=[
            pl.BlockSpec((G, HC), lambda i: (0, 0)),
            pl.BlockSpec((1, G), lambda i: (0, 0)),
        ],
        out_shape=[
            jax.ShapeDtypeStruct((G, HC), jnp.float32),
            jax.ShapeDtypeStruct((1, G), jnp.float32),
        ],
    )(h, onehot, bias.reshape(1, HC))
    return out


NWORK = 32
EPW = E // NWORK
ABLK = 40
NEG = -1e30

_SC_MESH = dict(core_axis_name="c", subcore_axis_name="s")

_SC_PARAMS = pltpu.CompilerParams()
if "needs_layout_passes" in pltpu.CompilerParams.__dataclass_fields__:
    _SC_PARAMS = dataclasses.replace(_SC_PARAMS, needs_layout_passes=False)


def _alpha_body(xl_hbm, xr_hbm, e_hbm, src_hbm, dst_hbm, att_hbm,
                alpha_hbm, tmax_hbm,
                sidx_v, didx_v, xl_v0, xr_v0, xl_v1, xr_v1, e_v0,
                alpha_v, att_v, maxv_v, semA, semB):
    wid = lax.axis_index("s") * 2 + lax.axis_index("c")
    base_t = wid * EPW
    pltpu.sync_copy(att_hbm, att_v)
    pltpu.sync_copy(src_hbm.at[pl.ds(base_t, EPW)],
                    sidx_v.at[pl.ds(0, EPW)])
    pltpu.sync_copy(dst_hbm.at[pl.ds(base_t, EPW)],
                    didx_v.at[pl.ds(0, EPW)])
    sidx_v[pl.ds(EPW, 16)] = jnp.zeros((16,), jnp.int32)
    sidx_v[pl.ds(EPW + 16, 16)] = jnp.zeros((16,), jnp.int32)
    didx_v[pl.ds(EPW, 16)] = jnp.zeros((16,), jnp.int32)
    didx_v[pl.ds(EPW + 16, 16)] = jnp.zeros((16,), jnp.int32)
    maxv_v[...] = jnp.full((16,), NEG, jnp.float32)
    lane = lax.iota(jnp.int32, 16)

    def issue(b, xl_v, xr_v, sem):
        pltpu.async_copy(xl_hbm.at[sidx_v.at[pl.ds(b * ABLK, ABLK)]],
                         xl_v, sem)
        pltpu.async_copy(xr_hbm.at[didx_v.at[pl.ds(b * ABLK, ABLK)]],
                         xr_v, sem)

    def wait(xl_v, xr_v, sem):
        pltpu.make_async_copy(e_hbm.at[pl.ds(0, ABLK)], xl_v, sem).wait()
        pltpu.make_async_copy(e_hbm.at[pl.ds(0, ABLK)], xr_v, sem).wait()

    def compute(b, xl_v, xr_v):
        pltpu.sync_copy(e_hbm.at[pl.ds(base_t + b * ABLK, ABLK)], e_v0)
        e_v = e_v0
        @pl.loop(0, ABLK)
        def _edge(j):
            row = jnp.full((16,), NEG, jnp.float32)
            for h in range(H):
                acc = jnp.zeros((16,), jnp.float32)
                for q in range(C // 16):
                    sl = pl.ds(h * C + q * 16, 16)
                    m = xl_v[j, sl] + xr_v[j, sl] + e_v[j, sl]
                    m = jnp.where(m > 0, m, NEG_SLOPE * m)
                    acc = acc + m * att_v[h, pl.ds(q * 16, 16)]
                row = jnp.where(lane == h, jnp.sum(acc), row)
            alpha_v[j, pl.ds(0, 16)] = row

        @pl.loop(0, ABLK)
        def _mx(j):
            maxv_v[...] = jnp.maximum(maxv_v[...], alpha_v[j, pl.ds(0, 16)])

        pltpu.sync_copy(alpha_v, alpha_hbm.at[pl.ds(base_t + b * ABLK, ABLK)])

    issue(0, xl_v0, xr_v0, semA)

    @pl.loop(0, EPW // ABLK // 2)
    def _pair(p):
        issue(2 * p + 1, xl_v1, xr_v1, semB)
        wait(xl_v0, xr_v0, semA)
        compute(2 * p, xl_v0, xr_v0)
        issue(2 * p + 2, xl_v0, xr_v0, semA)
        wait(xl_v1, xr_v1, semB)
        compute(2 * p + 1, xl_v1, xr_v1)

    wait(xl_v0, xr_v0, semA)
    compute(EPW // ABLK - 1, xl_v0, xr_v0)
    pltpu.sync_copy(maxv_v, tmax_hbm.at[wid])


def _sc_alpha(xl, xr, e, src, dst, att):
    kern = functools.partial(
        pl.kernel,
        mesh=plsc.VectorSubcoreMesh(**_SC_MESH),
        compiler_params=_SC_PARAMS,
        out_type=[
            jax.ShapeDtypeStruct((E, 128), jnp.float32),
            jax.ShapeDtypeStruct((NWORK, 16), jnp.float32),
        ],
        scratch_types=[
            pltpu.VMEM((EPW + 2 * ABLK,), jnp.int32),
            pltpu.VMEM((EPW + 2 * ABLK,), jnp.int32),
            pltpu.VMEM((ABLK, HC), jnp.float32),
            pltpu.VMEM((ABLK, HC), jnp.float32),
            pltpu.VMEM((ABLK, HC), jnp.float32),
            pltpu.VMEM((ABLK, HC), jnp.float32),
            pltpu.VMEM((ABLK, HC), jnp.float32),
            pltpu.VMEM((ABLK, 128), jnp.float32),
            pltpu.VMEM((H, C), jnp.float32),
            pltpu.VMEM((16,), jnp.float32),
            pltpu.SemaphoreType.DMA,
            pltpu.SemaphoreType.DMA,
        ],
    )(_alpha_body)
    return kern(xl, xr, e, src, dst, att)


NB = 256
BN = 40
TBL = E + 16
SLC1 = 5008
SLC0 = E - (NWORK - 1) * SLC1
BL0 = SLC0 // 16
BL1 = SLC1 // 16


def _hist_body(dst_hbm, hist_hbm, dstb_v, hist_v):
    wid = lax.axis_index("s") * 2 + lax.axis_index("c")
    base = wid * SLC1
    pltpu.sync_copy(dst_hbm.at[pl.ds(base, SLC0)], dstb_v.at[pl.ds(0, SLC0)])

    @pl.when(wid < NWORK - 1)
    def _ld_tail():
        pltpu.sync_copy(dst_hbm.at[pl.ds(base + SLC0, SLC1 - SLC0)],
                        dstb_v.at[pl.ds(SLC0, SLC1 - SLC0)])

    lane = lax.iota(jnp.int32, 16)
    zi = jnp.zeros((16,), jnp.int32)
    for i in range(NB // 16):
        hist_v[pl.ds(i * 16, 16)] = zi

    def _blk(b):
        bv = dstb_v[pl.ds(b * 16, 16)] // BN
        for l in range(16):
            b_l = bv[l]
            fb = (b_l // 16) * 16
            w = hist_v[pl.ds(fb, 16)]
            hist_v[pl.ds(fb, 16)] = w + jnp.where(lane == b_l - fb, 1, 0)

    pl.loop(0, BL0)(_blk)

    @pl.when(wid < NWORK - 1)
    def _tail():
        pl.loop(BL0, BL1)(_blk)

    pltpu.sync_copy(hist_v, hist_hbm.at[wid])


def _sc_hist(dst):
    kern = functools.partial(
        pl.kernel,
        mesh=plsc.VectorSubcoreMesh(**_SC_MESH),
        compiler_params=_SC_PARAMS,
        out_type=jax.ShapeDtypeStruct((NWORK, NB), jnp.int32),
        scratch_types=[
            pltpu.VMEM((SLC1,), jnp.int32),
            pltpu.VMEM((NB,), jnp.int32),
        ],
    )(_hist_body)
    return kern(dst)


def _sort_body(src_hbm, dst_hbm, hist_hbm, table_hbm, bst_hbm,
               srcb_v, dstb_v, h32_v, offs_v, trip_v, pos_v, bst_v, sem):
    wid = lax.axis_index("s") * 2 + lax.axis_index("c")
    base = wid * SLC1
    pltpu.sync_copy(src_hbm.at[pl.ds(base, SLC0)], srcb_v.at[pl.ds(0, SLC0)])
    pltpu.sync_copy(dst_hbm.at[pl.ds(base, SLC0)], dstb_v.at[pl.ds(0, SLC0)])

    @pl.when(wid < NWORK - 1)
    def _ld_tail():
        pltpu.sync_copy(src_hbm.at[pl.ds(base + SLC0, SLC1 - SLC0)],
                        srcb_v.at[pl.ds(SLC0, SLC1 - SLC0)])
        pltpu.sync_copy(dst_hbm.at[pl.ds(base + SLC0, SLC1 - SLC0)],
                        dstb_v.at[pl.ds(SLC0, SLC1 - SLC0)])

    pltpu.sync_copy(hist_hbm, h32_v)
    lane = lax.iota(jnp.int32, 16)

    run = jnp.int32(0)
    for i in range(NB // 16):
        tot = jnp.zeros((16,), jnp.int32)
        pre = jnp.zeros((16,), jnp.int32)
        for w in range(NWORK):
            row = h32_v[w, pl.ds(i * 16, 16)]
            tot = tot + row
            pre = pre + jnp.where(wid > w, row, 0)
        cs = plsc.cumsum(tot)
        excl = cs - tot
        offs_v[pl.ds(i * 16, 16)] = run + excl + pre
        startv = run + excl
        endv = run + cs
        for l in range(16):
            bst_v[i * 16 + l, pl.ds(0, 16)] = (
                jnp.where(lane == 0, startv[l], 0)
                + jnp.where(lane == 1, endv[l], 0)
            )
        run = run + cs[15]

    @pl.when(wid == 0)
    def _aux():
        pltpu.sync_copy(bst_v, bst_hbm)

        @pl.loop(0, 16)
        def _sr(j):
            trip_v[j, pl.ds(0, 16)] = jnp.where(lane == 1, -1, 0)

        pltpu.sync_copy(trip_v, table_hbm.at[pl.ds(E, 16)])

    def _blk(b):
        dv = dstb_v[pl.ds(b * 16, 16)]
        sv = srcb_v[pl.ds(b * 16, 16)]
        bv = dv // BN
        eidv = base + b * 16 + lane
        posacc = jnp.zeros((16,), jnp.int32)
        for l in range(16):
            b_l = bv[l]
            fb = (b_l // 16) * 16
            w = offs_v[pl.ds(fb, 16)]
            sel = lane == (b_l - fb)
            pos_l = jnp.max(jnp.where(sel, w, -1))
            offs_v[pl.ds(fb, 16)] = w + jnp.where(sel, 1, 0)
            posacc = jnp.where(lane == l, pos_l, posacc)
            trip_v[l, pl.ds(0, 16)] = (
                jnp.where(lane == 0, sv[l], 0)
                + jnp.where(lane == 1, dv[l], 0)
                + jnp.where(lane == 2, eidv[l], 0)
            )
        pos_v[0, pl.ds(0, 16)] = posacc
        pltpu.async_copy(trip_v, table_hbm.at[pos_v.at[0]], sem).wait()

    pl.loop(0, BL0)(_blk)

    @pl.when(wid < NWORK - 1)
    def _tail():
        pl.loop(BL0, BL1)(_blk)


def _sc_sort(src, dst, hist):
    kern = functools.partial(
        pl.kernel,
        mesh=plsc.VectorSubcoreMesh(**_SC_MESH),
        compiler_params=_SC_PARAMS,
        out_type=[
            jax.ShapeDtypeStruct((TBL, 128), jnp.int32),
            jax.ShapeDtypeStruct((NB, 16), jnp.int32),
        ],
        scratch_types=[
            pltpu.VMEM((SLC1,), jnp.int32),
            pltpu.VMEM((SLC1,), jnp.int32),
            pltpu.VMEM((NWORK, NB), jnp.int32),
            pltpu.VMEM((NB,), jnp.int32),
            pltpu.VMEM((16, 128), jnp.int32),
            pltpu.VMEM((1, 16), jnp.int32),
            pltpu.VMEM((NB, 16), jnp.int32),
            pltpu.SemaphoreType.DMA,
        ],
    )(_sort_body)
    return kern(src, dst, hist)


def _agg_body(alpha_hbm, xl_hbm, table_hbm, bst_hbm, tmax_hbm, out_hbm,
              bst_v, trip_v, eix_v, six_v, axr_v, xlr_v, acc_v, den_v,
              tmax_v, sem):
    cid = lax.axis_index("c")
    sid = lax.axis_index("s")
    pltpu.sync_copy(tmax_hbm, tmax_v)
    pltpu.sync_copy(bst_hbm, bst_v)
    mrow = jnp.full((16,), NEG, jnp.float32)
    for w in range(NWORK):
        mrow = jnp.maximum(mrow, tmax_v[w, pl.ds(0, 16)])
    gvreg = jnp.zeros((16,), jnp.float32) + jnp.max(mrow)
    lane = lax.iota(jnp.int32, 16)
    zf = jnp.zeros((16,), jnp.float32)

    @pl.loop(0, NB // NWORK)
    def _bk(k):
        b = (cid * (NB // NWORK) + k) * 16 + sid
        brow = bst_v[b, pl.ds(0, 16)]
        start = brow[0]
        end = brow[1]
        lo = b * BN

        @pl.loop(0, BN)
        def _z(r):
            for q in range(HC // 16):
                acc_v[r, pl.ds(q * 16, 16)] = zf
            den_v[r, pl.ds(0, 16)] = zf

        a0 = (start // 16) * 16

        @pl.loop(0, (end - a0 + 15) // 16)
        def _blk(g):
            pltpu.sync_copy(table_hbm.at[pl.ds(a0 + g * 16, 16)], trip_v)
            ei = jnp.zeros((16,), jnp.int32)
            si = jnp.zeros((16,), jnp.int32)
            for l in range(16):
                tr = trip_v[l, pl.ds(0, 16)]
                ei = jnp.where(lane == l, tr[2], ei)
                si = jnp.where(lane == l, tr[0], si)
            eix_v[...] = ei
            six_v[...] = si
            c1 = pltpu.async_copy(alpha_hbm.at[eix_v], axr_v, sem)
            c2 = pltpu.async_copy(xl_hbm.at[six_v], xlr_v, sem)
            c1.wait()
            c2.wait()

            @pl.loop(0, 16)
            def _edge(j):
                dl = trip_v[j, pl.ds(0, 16)][1] - lo

                @pl.when((dl >= 0) & (dl < BN))
                def _proc():
                    ex = jnp.exp(axr_v[j, pl.ds(0, 16)] - gvreg)
                    den_v[dl, pl.ds(0, 16)] = den_v[dl, pl.ds(0, 16)] + ex
                    for h in range(H):
                        a_s = ex[h]
                        for q in range(C // 16):
                            sl = pl.ds(h * C + q * 16, 16)
                            acc_v[dl, sl] = (acc_v[dl, sl]
                                             + xlr_v[j, sl] * a_s)

        @pl.when(lo < N)
        def _dump():
            @pl.loop(0, BN)
            def _n(r):
                invv = 1.0 / (den_v[r, pl.ds(0, 16)] + 1e-16)
                for h in range(H):
                    iv = invv[h]
                    for q in range(C // 16):
                        sl = pl.ds(h * C + q * 16, 16)
                        acc_v[r, sl] = acc_v[r, sl] * iv

            pltpu.sync_copy(acc_v, out_hbm.at[pl.ds(lo, BN)])


def _sc_aggregate(alpha128, xl, table, bst, tmax):
    kern = functools.partial(
        pl.kernel,
        mesh=plsc.VectorSubcoreMesh(**_SC_MESH),
        compiler_params=_SC_PARAMS,
        out_type=jax.ShapeDtypeStruct((N, HC), jnp.float32),
        scratch_types=[
            pltpu.VMEM((NB, 16), jnp.int32),
            pltpu.VMEM((16, 128), jnp.int32),
            pltpu.VMEM((16,), jnp.int32),
            pltpu.VMEM((16,), jnp.int32),
            pltpu.VMEM((16, 128), jnp.float32),
            pltpu.VMEM((16, HC), jnp.float32),
            pltpu.VMEM((BN, HC), jnp.float32),
            pltpu.VMEM((BN, 16), jnp.float32),
            pltpu.VMEM((NWORK, 16), jnp.float32),
            pltpu.SemaphoreType.DMA,
        ],
    )(_agg_body)
    return kern(alpha128, xl, table, bst, tmax)


def _gat_layer(x, src, dst, e, table, bst, Wl, bl, Wr, br, att):
    xl, xr = _dual_project(x, Wl, bl, Wr, br)
    alpha128, tmax = _sc_alpha(xl, xr, e, src, dst, att)
    return _sc_aggregate(alpha128, xl, table, bst, tmax)


def kernel(x, edge_index, edge_attr, batch, Wl1, bl1, Wr1, br1, We1, be1,
           att1, bias1, Wl2, bl2, Wr2, br2, We2, be2, att2, bias2):
    src = edge_index[0]
    dst = edge_index[1]
    hist = _sc_hist(dst)
    table, bst = _sc_sort(src, dst, hist)
    e1 = _edge_project(edge_attr, We1, be1)
    e2 = _edge_project(edge_attr, We2, be2)
    h = _gat_layer(x, src, dst, e1, table, bst, Wl1, bl1, Wr1, br1, att1)
    bl2f = bias1 @ Wl2 + bl2
    br2f = bias1 @ Wr2 + br2
    h2 = _gat_layer(h, src, dst, e2, table, bst, Wl2, bl2f, Wr2, br2f, att2)
    return _mean_pool(h2, batch, bias2)
